# baseline (speedup 1.0000x reference)
import functools

import jax
import jax.numpy as jnp
from jax import lax
from jax.experimental import pallas as pl

B = 32
M = 862
L_SEQ = 96
D_MODEL = 128
N_HEADS = 8
D_FF = 256
K_NN = 16
N_LAYERS = 2
DH = D_MODEL // N_HEADS


def _topk_body(x_ref, nbr_ref):
    x = x_ref[0]
    mean = jnp.mean(x, axis=0)
    c = x - mean[None, :]
    s2 = lax.dot_general(c, c, (((0,), (0,)), ((), ())),
                         preferred_element_type=jnp.float32)
    cov = s2 * (1.0 / (L_SEQ - 1))
    dvar = jnp.sum(c * c, axis=0) * (1.0 / (L_SEQ - 1))
    std = jnp.sqrt(dvar)
    stdw = jnp.where(std == 0.0, 1.0, std)
    sim = cov / (stdw[:, None] * stdw[None, :])

    iota_l = lax.broadcasted_iota(jnp.int32, (M, M), 1)
    cur = sim
    for k in range(K_NN + 1):
        idx = jnp.argmin(cur, axis=1).astype(jnp.int32)
        if k > 0:
            nbr_ref[0, k - 1, :] = idx
        if k < K_NN:
            cur = jnp.where(iota_l == idx[:, None], jnp.inf, cur)


def _ln(x, g, b):
    mu = jnp.mean(x, axis=-1, keepdims=True)
    xc = x - mu
    var = jnp.mean(xc * xc, axis=-1, keepdims=True)
    return xc * lax.rsqrt(var + 1e-5) * g[None, :] + b[None, :]


def _main_body(dst_ref, x_ref,
               W1_ref, b1_ref, W2_ref, b2_ref,
               Wq_ref, bq_ref, Wk_ref, bk_ref, Wv_ref, bv_ref,
               Wo_ref, bo_ref, ln1g_ref, ln1b_ref,
               Wf1_ref, bf1_ref, Wf2_ref, bf2_ref, ln2g_ref, ln2b_ref,
               out_ref):
    dstm = dst_ref[0]
    iota_d = lax.broadcasted_iota(jnp.int32, (M, M), 0)
    A = jnp.zeros((M, M), jnp.float32)
    for t in range(K_NN):
        A = A + (dstm[t, :][None, :] == iota_d).astype(jnp.float32)
    deg = 1.0 + jnp.sum(A, axis=1)
    dis = lax.rsqrt(deg)
    iota_s = lax.broadcasted_iota(jnp.int32, (M, M), 1)
    eye = (iota_d == iota_s).astype(jnp.float32)
    Ahat = (A + eye) * (dis[:, None] * dis[None, :])

    x0 = x_ref[0]

    def mm(a, b):
        return jnp.dot(a, b, preferred_element_type=jnp.float32)

    z1 = mm(x0, W1_ref[...])
    x1 = jnp.maximum(mm(Ahat, z1) + b1_ref[...][None, :], 0.0)
    z2 = mm(x1, W2_ref[...])
    xg = jnp.maximum(mm(Ahat, z2) + b2_ref[...][None, :], 0.0)

    h = x0
    for l in range(N_LAYERS):
        q = mm(h, Wq_ref[l]) + bq_ref[l][None, :]
        k = mm(xg, Wk_ref[l]) + bk_ref[l][None, :]
        v = mm(xg, Wv_ref[l]) + bv_ref[l][None, :]
        ones_col = jnp.ones((M, 1), jnp.float32)
        ohs = []
        for hd in range(N_HEADS):
            qh = q[:, hd * DH:(hd + 1) * DH]
            kh = k[:, hd * DH:(hd + 1) * DH]
            vh = v[:, hd * DH:(hd + 1) * DH]
            s = lax.dot_general(qh.astype(jnp.bfloat16), kh.astype(jnp.bfloat16),
                                (((1,), (1,)), ((), ())),
                                preferred_element_type=jnp.float32)
            e = jnp.exp(s).astype(jnp.bfloat16)
            vh1 = jnp.concatenate([vh, ones_col], axis=1).astype(jnp.bfloat16)
            oh_ext = mm(e, vh1)
            ohs.append(oh_ext[:, :DH] * (1.0 / oh_ext[:, DH])[:, None])
        o = jnp.concatenate(ohs, axis=1)
        a = mm(o, Wo_ref[l]) + bo_ref[l][None, :]
        h = _ln(h + a, ln1g_ref[l], ln1b_ref[l])
        ff = jnp.maximum(mm(h, Wf1_ref[l]) + bf1_ref[l][None, :], 0.0)
        ff = mm(ff, Wf2_ref[l]) + bf2_ref[l][None, :]
        h = _ln(h + ff, ln2g_ref[l], ln2b_ref[l])
    out_ref[0] = h


def _full(whole):
    return pl.BlockSpec(whole, lambda b: (0,) * len(whole))


def kernel(enc_out_vari, x_enc, W1, b1, W2, b2, Wq, bq, Wk, bk, Wv, bv,
           Wo, bo, ln1_g, ln1_b, Wf1, bf1, Wf2, bf2, ln2_g, ln2_b):
    nbrs_t = pl.pallas_call(
        _topk_body,
        grid=(B,),
        in_specs=[pl.BlockSpec((1, L_SEQ, M), lambda b: (b, 0, 0))],
        out_specs=pl.BlockSpec((1, K_NN, M), lambda b: (b, 0, 0)),
        out_shape=jax.ShapeDtypeStruct((B, K_NN, M), jnp.int32),
    )(x_enc)

    dst = nbrs_t.transpose(0, 2, 1).reshape(B, K_NN, M)

    inv_sqrt_dh = 1.0 / (DH ** 0.5)
    Wq = Wq * inv_sqrt_dh
    bq = bq * inv_sqrt_dh

    out = pl.pallas_call(
        _main_body,
        grid=(B,),
        in_specs=[
            pl.BlockSpec((1, K_NN, M), lambda b: (b, 0, 0)),
            pl.BlockSpec((1, M, D_MODEL), lambda b: (b, 0, 0)),
            _full(W1.shape), _full(b1.shape), _full(W2.shape), _full(b2.shape),
            _full(Wq.shape), _full(bq.shape), _full(Wk.shape), _full(bk.shape),
            _full(Wv.shape), _full(bv.shape), _full(Wo.shape), _full(bo.shape),
            _full(ln1_g.shape), _full(ln1_b.shape),
            _full(Wf1.shape), _full(bf1.shape),
            _full(Wf2.shape), _full(bf2.shape),
            _full(ln2_g.shape), _full(ln2_b.shape),
        ],
        out_specs=pl.BlockSpec((1, M, D_MODEL), lambda b: (b, 0, 0)),
        out_shape=jax.ShapeDtypeStruct((B, M, D_MODEL), jnp.float32),
    )(dst, enc_out_vari, W1, b1, W2, b2, Wq, bq, Wk, bk, Wv, bv,
      Wo, bo, ln1_g, ln1_b, Wf1, bf1, Wf2, bf2, ln2_g, ln2_b)
    return out


# speedup vs baseline: 1.0586x; 1.0586x over previous
import functools

import jax
import jax.numpy as jnp
from jax import lax
from jax.experimental import pallas as pl

B = 32
M = 862
L_SEQ = 96
D_MODEL = 128
N_HEADS = 8
D_FF = 256
K_NN = 16
N_LAYERS = 2
DH = D_MODEL // N_HEADS


def _topk_body(x_ref, nbr_ref):
    x = x_ref[0]
    mean = jnp.mean(x, axis=0)
    c = x - mean[None, :]
    s2 = lax.dot_general(c, c, (((0,), (0,)), ((), ())),
                         preferred_element_type=jnp.float32)
    cov = s2 * (1.0 / (L_SEQ - 1))
    dvar = jnp.sum(c * c, axis=0) * (1.0 / (L_SEQ - 1))
    std = jnp.sqrt(dvar)
    stdw = jnp.where(std == 0.0, 1.0, std)
    sim = cov / (stdw[:, None] * stdw[None, :])

    iota_l = lax.broadcasted_iota(jnp.int32, (M, M), 1)
    cur = sim
    for k in range(K_NN + 1):
        idx = jnp.argmin(cur, axis=1).astype(jnp.int32)
        if k > 0:
            nbr_ref[0, k - 1, :] = idx
        if k < K_NN:
            cur = jnp.where(iota_l == idx[:, None], jnp.inf, cur)


def _ln(x, g, b):
    mu = jnp.mean(x, axis=-1, keepdims=True)
    xc = x - mu
    var = jnp.mean(xc * xc, axis=-1, keepdims=True)
    return xc * lax.rsqrt(var + 1e-5) * g[None, :] + b[None, :]


def _main_body(dst_ref, x_ref,
               W1_ref, b1_ref, W2_ref, b2_ref,
               Wq_ref, bq_ref, Wk_ref, bk_ref, Wv_ref, bv_ref,
               Wo_ref, bo_ref, ln1g_ref, ln1b_ref,
               Wf1_ref, bf1_ref, Wf2_ref, bf2_ref, ln2g_ref, ln2b_ref,
               out_ref):
    dstm = dst_ref[0]
    iota_d = lax.broadcasted_iota(jnp.int32, (M, M), 0)
    iota16 = iota_d.astype(jnp.int16)
    dstm16 = dstm.astype(jnp.int16)
    A16 = jnp.zeros((M, M), jnp.int16)
    for t in range(K_NN):
        A16 = A16 + (dstm16[t, :][None, :] == iota16).astype(jnp.int16)
    A = A16.astype(jnp.float32)
    deg = 1.0 + jnp.sum(A, axis=1)
    dis = lax.rsqrt(deg)
    iota_s = lax.broadcasted_iota(jnp.int32, (M, M), 1)
    eye = (iota_d == iota_s).astype(jnp.float32)
    Ahat = (A + eye) * (dis[:, None] * dis[None, :])

    x0 = x_ref[0]

    def mm(a, b):
        return jnp.dot(a, b, preferred_element_type=jnp.float32)

    z1 = mm(x0, W1_ref[...])
    x1 = jnp.maximum(mm(Ahat, z1) + b1_ref[...][None, :], 0.0)
    z2 = mm(x1, W2_ref[...])
    xg = jnp.maximum(mm(Ahat, z2) + b2_ref[...][None, :], 0.0)

    h = x0
    for l in range(N_LAYERS):
        q = mm(h, Wq_ref[l]) + bq_ref[l][None, :]
        k = mm(xg, Wk_ref[l]) + bk_ref[l][None, :]
        v = mm(xg, Wv_ref[l]) + bv_ref[l][None, :]
        ones_col = jnp.ones((M, 1), jnp.float32)
        ohs = []
        for hd in range(N_HEADS):
            qh = q[:, hd * DH:(hd + 1) * DH]
            kh = k[:, hd * DH:(hd + 1) * DH]
            vh = v[:, hd * DH:(hd + 1) * DH]
            s = lax.dot_general(qh.astype(jnp.bfloat16), kh.astype(jnp.bfloat16),
                                (((1,), (1,)), ((), ())),
                                preferred_element_type=jnp.float32)
            e = jnp.exp(s)
            vh1 = jnp.concatenate([vh, ones_col], axis=1)
            oh_ext = mm(e, vh1)
            ohs.append(oh_ext[:, :DH] * (1.0 / oh_ext[:, DH])[:, None])
        o = jnp.concatenate(ohs, axis=1)
        a = mm(o, Wo_ref[l]) + bo_ref[l][None, :]
        h = _ln(h + a, ln1g_ref[l], ln1b_ref[l])
        ff = jnp.maximum(mm(h, Wf1_ref[l]) + bf1_ref[l][None, :], 0.0)
        ff = mm(ff, Wf2_ref[l]) + bf2_ref[l][None, :]
        h = _ln(h + ff, ln2g_ref[l], ln2b_ref[l])
    out_ref[0] = h


def _full(whole):
    return pl.BlockSpec(whole, lambda b: (0,) * len(whole))


def kernel(enc_out_vari, x_enc, W1, b1, W2, b2, Wq, bq, Wk, bk, Wv, bv,
           Wo, bo, ln1_g, ln1_b, Wf1, bf1, Wf2, bf2, ln2_g, ln2_b):
    nbrs_t = pl.pallas_call(
        _topk_body,
        grid=(B,),
        in_specs=[pl.BlockSpec((1, L_SEQ, M), lambda b: (b, 0, 0))],
        out_specs=pl.BlockSpec((1, K_NN, M), lambda b: (b, 0, 0)),
        out_shape=jax.ShapeDtypeStruct((B, K_NN, M), jnp.int32),
    )(x_enc)

    dst = nbrs_t.transpose(0, 2, 1).reshape(B, K_NN, M)

    inv_sqrt_dh = 1.0 / (DH ** 0.5)
    Wq = Wq * inv_sqrt_dh
    bq = bq * inv_sqrt_dh

    out = pl.pallas_call(
        _main_body,
        grid=(B,),
        in_specs=[
            pl.BlockSpec((1, K_NN, M), lambda b: (b, 0, 0)),
            pl.BlockSpec((1, M, D_MODEL), lambda b: (b, 0, 0)),
            _full(W1.shape), _full(b1.shape), _full(W2.shape), _full(b2.shape),
            _full(Wq.shape), _full(bq.shape), _full(Wk.shape), _full(bk.shape),
            _full(Wv.shape), _full(bv.shape), _full(Wo.shape), _full(bo.shape),
            _full(ln1_g.shape), _full(ln1_b.shape),
            _full(Wf1.shape), _full(bf1.shape),
            _full(Wf2.shape), _full(bf2.shape),
            _full(ln2_g.shape), _full(ln2_b.shape),
        ],
        out_specs=pl.BlockSpec((1, M, D_MODEL), lambda b: (b, 0, 0)),
        out_shape=jax.ShapeDtypeStruct((B, M, D_MODEL), jnp.float32),
    )(dst, enc_out_vari, W1, b1, W2, b2, Wq, bq, Wk, bk, Wv, bv,
      Wo, bo, ln1_g, ln1_b, Wf1, bf1, Wf2, bf2, ln2_g, ln2_b)
    return out


# speedup vs baseline: 1.0608x; 1.0021x over previous
import functools

import jax
import jax.numpy as jnp
from jax import lax
from jax.experimental import pallas as pl

B = 32
M = 862
L_SEQ = 96
D_MODEL = 128
N_HEADS = 8
D_FF = 256
K_NN = 16
N_LAYERS = 2
DH = D_MODEL // N_HEADS


def _topk_body(x_ref, nbr_ref):
    x = x_ref[0]
    mean = jnp.mean(x, axis=0)
    c = x - mean[None, :]
    s2 = lax.dot_general(c, c, (((0,), (0,)), ((), ())),
                         preferred_element_type=jnp.float32)
    cov = s2 * (1.0 / (L_SEQ - 1))
    dvar = jnp.sum(c * c, axis=0) * (1.0 / (L_SEQ - 1))
    std = jnp.sqrt(dvar)
    inv = 1.0 / jnp.where(std == 0.0, 1.0, std)
    sim = cov * (inv[:, None] * inv[None, :])

    iota_l = lax.broadcasted_iota(jnp.int32, (M, M), 1)
    cur = sim
    for k in range(K_NN + 1):
        idx = jnp.argmin(cur, axis=1).astype(jnp.int32)
        if k > 0:
            nbr_ref[0, k - 1, :] = idx
        if k < K_NN:
            cur = jnp.where(iota_l == idx[:, None], jnp.inf, cur)


def _ln(x, g, b):
    mu = jnp.mean(x, axis=-1, keepdims=True)
    xc = x - mu
    var = jnp.mean(xc * xc, axis=-1, keepdims=True)
    return xc * lax.rsqrt(var + 1e-5) * g[None, :] + b[None, :]


def _main_body(dst_ref, x_ref,
               W1_ref, b1_ref, W2_ref, b2_ref,
               Wq_ref, bq_ref, Wk_ref, bk_ref, Wv_ref, bv_ref,
               Wo_ref, bo_ref, ln1g_ref, ln1b_ref,
               Wf1_ref, bf1_ref, Wf2_ref, bf2_ref, ln2g_ref, ln2b_ref,
               out_ref):
    dstm = dst_ref[0]
    iota_d = lax.broadcasted_iota(jnp.int32, (M, M), 0)
    iota16 = iota_d.astype(jnp.int16)
    dstm16 = dstm.astype(jnp.int16)
    A16 = jnp.zeros((M, M), jnp.int16)
    for t in range(K_NN):
        A16 = A16 + (dstm16[t, :][None, :] == iota16).astype(jnp.int16)
    A = A16.astype(jnp.float32)
    deg = 1.0 + jnp.sum(A, axis=1)
    dis = lax.rsqrt(deg)
    iota_s = lax.broadcasted_iota(jnp.int32, (M, M), 1)
    eye = (iota_d == iota_s).astype(jnp.float32)
    Ahat = (A + eye) * (dis[:, None] * dis[None, :])

    x0 = x_ref[0]

    def mm(a, b):
        return jnp.dot(a, b, preferred_element_type=jnp.float32)

    z1 = mm(x0, W1_ref[...])
    x1 = jnp.maximum(mm(Ahat, z1) + b1_ref[...][None, :], 0.0)
    z2 = mm(x1, W2_ref[...])
    xg = jnp.maximum(mm(Ahat, z2) + b2_ref[...][None, :], 0.0)

    h = x0
    for l in range(N_LAYERS):
        q = mm(h, Wq_ref[l]) + bq_ref[l][None, :]
        k = mm(xg, Wk_ref[l]) + bk_ref[l][None, :]
        v = mm(xg, Wv_ref[l]) + bv_ref[l][None, :]
        ones_col = jnp.ones((M, 1), jnp.float32)
        ohs = []
        for hd in range(N_HEADS):
            qh = q[:, hd * DH:(hd + 1) * DH]
            kh = k[:, hd * DH:(hd + 1) * DH]
            vh = v[:, hd * DH:(hd + 1) * DH]
            s = lax.dot_general(qh.astype(jnp.bfloat16), kh.astype(jnp.bfloat16),
                                (((1,), (1,)), ((), ())),
                                preferred_element_type=jnp.float32)
            e = jnp.exp(s)
            vh1 = jnp.concatenate([vh, ones_col], axis=1)
            oh_ext = mm(e, vh1)
            ohs.append(oh_ext[:, :DH] * (1.0 / oh_ext[:, DH])[:, None])
        o = jnp.concatenate(ohs, axis=1)
        a = mm(o, Wo_ref[l]) + bo_ref[l][None, :]
        h = _ln(h + a, ln1g_ref[l], ln1b_ref[l])
        ff = jnp.maximum(mm(h, Wf1_ref[l]) + bf1_ref[l][None, :], 0.0)
        ff = mm(ff, Wf2_ref[l]) + bf2_ref[l][None, :]
        h = _ln(h + ff, ln2g_ref[l], ln2b_ref[l])
    out_ref[0] = h


def _full(whole):
    return pl.BlockSpec(whole, lambda b: (0,) * len(whole))


def kernel(enc_out_vari, x_enc, W1, b1, W2, b2, Wq, bq, Wk, bk, Wv, bv,
           Wo, bo, ln1_g, ln1_b, Wf1, bf1, Wf2, bf2, ln2_g, ln2_b):
    nbrs_t = pl.pallas_call(
        _topk_body,
        grid=(B,),
        in_specs=[pl.BlockSpec((1, L_SEQ, M), lambda b: (b, 0, 0))],
        out_specs=pl.BlockSpec((1, K_NN, M), lambda b: (b, 0, 0)),
        out_shape=jax.ShapeDtypeStruct((B, K_NN, M), jnp.int32),
    )(x_enc)

    dst = nbrs_t.transpose(0, 2, 1).reshape(B, K_NN, M)

    inv_sqrt_dh = 1.0 / (DH ** 0.5)
    Wq = Wq * inv_sqrt_dh
    bq = bq * inv_sqrt_dh

    out = pl.pallas_call(
        _main_body,
        grid=(B,),
        in_specs=[
            pl.BlockSpec((1, K_NN, M), lambda b: (b, 0, 0)),
            pl.BlockSpec((1, M, D_MODEL), lambda b: (b, 0, 0)),
            _full(W1.shape), _full(b1.shape), _full(W2.shape), _full(b2.shape),
            _full(Wq.shape), _full(bq.shape), _full(Wk.shape), _full(bk.shape),
            _full(Wv.shape), _full(bv.shape), _full(Wo.shape), _full(bo.shape),
            _full(ln1_g.shape), _full(ln1_b.shape),
            _full(Wf1.shape), _full(bf1.shape),
            _full(Wf2.shape), _full(bf2.shape),
            _full(ln2_g.shape), _full(ln2_b.shape),
        ],
        out_specs=pl.BlockSpec((1, M, D_MODEL), lambda b: (b, 0, 0)),
        out_shape=jax.ShapeDtypeStruct((B, M, D_MODEL), jnp.float32),
    )(dst, enc_out_vari, W1, b1, W2, b2, Wq, bq, Wk, bk, Wv, bv,
      Wo, bo, ln1_g, ln1_b, Wf1, bf1, Wf2, bf2, ln2_g, ln2_b)
    return out


# speedup vs baseline: 1.0712x; 1.0098x over previous
import functools

import jax
import jax.numpy as jnp
from jax import lax
from jax.experimental import pallas as pl

B = 32
M = 862
L_SEQ = 96
D_MODEL = 128
N_HEADS = 8
D_FF = 256
K_NN = 16
N_LAYERS = 2
DH = D_MODEL // N_HEADS


def _topk_body(x_ref, nbr_ref):
    x = x_ref[0]
    mean = jnp.mean(x, axis=0)
    c = x - mean[None, :]
    s2 = lax.dot_general(c, c, (((0,), (0,)), ((), ())),
                         preferred_element_type=jnp.float32)
    cov = s2 * (1.0 / (L_SEQ - 1))
    dvar = jnp.sum(c * c, axis=0) * (1.0 / (L_SEQ - 1))
    std = jnp.sqrt(dvar)
    inv = 1.0 / jnp.where(std == 0.0, 1.0, std)
    sim = cov * (inv[:, None] * inv[None, :])

    iota_l = lax.broadcasted_iota(jnp.int32, (M, M), 1)
    cur = sim
    for k in range(K_NN + 1):
        idx = jnp.argmin(cur, axis=1).astype(jnp.int32)
        if k > 0:
            nbr_ref[0, k - 1, :] = idx
        if k < K_NN:
            cur = jnp.where(iota_l == idx[:, None], jnp.inf, cur)


def _ln(x, g, b):
    mu = jnp.mean(x, axis=-1, keepdims=True)
    xc = x - mu
    var = jnp.mean(xc * xc, axis=-1, keepdims=True)
    return xc * lax.rsqrt(var + 1e-5) * g[None, :] + b[None, :]


def _main_body(dst_ref, x_ref,
               W1_ref, b1_ref, W2_ref, b2_ref,
               Wq_ref, bq_ref, Wk_ref, bk_ref, Wv_ref, bv_ref,
               Wo_ref, bo_ref, ln1g_ref, ln1b_ref,
               Wf1_ref, bf1_ref, Wf2_ref, bf2_ref, ln2g_ref, ln2b_ref,
               out_ref):
    dstm = dst_ref[0]
    iota_d = lax.broadcasted_iota(jnp.int32, (M, M), 0)
    iota16 = iota_d.astype(jnp.int16)
    dstm16 = dstm.astype(jnp.int16)
    A16 = jnp.zeros((M, M), jnp.int16)
    for t in range(K_NN):
        A16 = A16 + (dstm16[t, :][None, :] == iota16).astype(jnp.int16)
    A = A16.astype(jnp.float32)
    deg = 1.0 + jnp.sum(A, axis=1)
    dis = lax.rsqrt(deg)

    x0 = x_ref[0]

    def mm(a, b):
        return jnp.dot(a, b, preferred_element_type=jnp.float32)

    z1 = mm(x0, W1_ref[...]) * dis[:, None]
    x1 = jnp.maximum((mm(A, z1) + z1) * dis[:, None] + b1_ref[...][None, :],
                     0.0)
    z2 = mm(x1, W2_ref[...]) * dis[:, None]
    xg = jnp.maximum((mm(A, z2) + z2) * dis[:, None] + b2_ref[...][None, :],
                     0.0)

    h = x0
    for l in range(N_LAYERS):
        q = mm(h, Wq_ref[l]) + bq_ref[l][None, :]
        k = mm(xg, Wk_ref[l]) + bk_ref[l][None, :]
        v = mm(xg, Wv_ref[l]) + bv_ref[l][None, :]
        ones_col = jnp.ones((M, 1), jnp.float32)
        ohs = []
        for hd in range(N_HEADS):
            qh = q[:, hd * DH:(hd + 1) * DH]
            kh = k[:, hd * DH:(hd + 1) * DH]
            vh = v[:, hd * DH:(hd + 1) * DH]
            s = lax.dot_general(qh.astype(jnp.bfloat16), kh.astype(jnp.bfloat16),
                                (((1,), (1,)), ((), ())),
                                preferred_element_type=jnp.float32)
            e = jnp.exp(s)
            vh1 = jnp.concatenate([vh, ones_col], axis=1)
            oh_ext = mm(e, vh1)
            ohs.append(oh_ext[:, :DH] * (1.0 / oh_ext[:, DH])[:, None])
        o = jnp.concatenate(ohs, axis=1)
        a = mm(o, Wo_ref[l]) + bo_ref[l][None, :]
        h = _ln(h + a, ln1g_ref[l], ln1b_ref[l])
        ff = jnp.maximum(mm(h, Wf1_ref[l]) + bf1_ref[l][None, :], 0.0)
        ff = mm(ff, Wf2_ref[l]) + bf2_ref[l][None, :]
        h = _ln(h + ff, ln2g_ref[l], ln2b_ref[l])
    out_ref[0] = h


def _full(whole):
    return pl.BlockSpec(whole, lambda b: (0,) * len(whole))


def kernel(enc_out_vari, x_enc, W1, b1, W2, b2, Wq, bq, Wk, bk, Wv, bv,
           Wo, bo, ln1_g, ln1_b, Wf1, bf1, Wf2, bf2, ln2_g, ln2_b):
    nbrs_t = pl.pallas_call(
        _topk_body,
        grid=(B,),
        in_specs=[pl.BlockSpec((1, L_SEQ, M), lambda b: (b, 0, 0))],
        out_specs=pl.BlockSpec((1, K_NN, M), lambda b: (b, 0, 0)),
        out_shape=jax.ShapeDtypeStruct((B, K_NN, M), jnp.int32),
    )(x_enc)

    dst = nbrs_t.transpose(0, 2, 1).reshape(B, K_NN, M)

    inv_sqrt_dh = 1.0 / (DH ** 0.5)
    Wq = Wq * inv_sqrt_dh
    bq = bq * inv_sqrt_dh

    out = pl.pallas_call(
        _main_body,
        grid=(B,),
        in_specs=[
            pl.BlockSpec((1, K_NN, M), lambda b: (b, 0, 0)),
            pl.BlockSpec((1, M, D_MODEL), lambda b: (b, 0, 0)),
            _full(W1.shape), _full(b1.shape), _full(W2.shape), _full(b2.shape),
            _full(Wq.shape), _full(bq.shape), _full(Wk.shape), _full(bk.shape),
            _full(Wv.shape), _full(bv.shape), _full(Wo.shape), _full(bo.shape),
            _full(ln1_g.shape), _full(ln1_b.shape),
            _full(Wf1.shape), _full(bf1.shape),
            _full(Wf2.shape), _full(bf2.shape),
            _full(ln2_g.shape), _full(ln2_b.shape),
        ],
        out_specs=pl.BlockSpec((1, M, D_MODEL), lambda b: (b, 0, 0)),
        out_shape=jax.ShapeDtypeStruct((B, M, D_MODEL), jnp.float32),
    )(dst, enc_out_vari, W1, b1, W2, b2, Wq, bq, Wk, bk, Wv, bv,
      Wo, bo, ln1_g, ln1_b, Wf1, bf1, Wf2, bf2, ln2_g, ln2_b)
    return out
